# Initial kernel scaffold; baseline (speedup 1.0000x reference)
import jax
import jax.numpy as jnp
from jax.experimental import pallas as pl


def kernel(latents, embedding_weight):
    raise NotImplementedError("write your pallas kernel here")


# speedup vs baseline: 38.3274x; 38.3274x over previous
import jax
import jax.numpy as jnp
from jax import lax
from jax.experimental import pallas as pl

K = 1024
D = 64
BETA = 0.25
BLK = 512


def _vq_block(flat_ref, emb_ref, out_ref, loss_ref):
    i = pl.program_id(0)
    x = flat_ref[...]
    e = emb_ref[...]
    m = lax.dot_general(x, e, (((1,), (1,)), ((), ())),
                        preferred_element_type=jnp.float32)
    sx = jnp.sum(x ** 2, axis=1, keepdims=True)
    se = jnp.sum(e ** 2, axis=1)
    dist = sx + se[None, :] - 2.0 * m
    minv = jnp.min(dist, axis=1, keepdims=True)
    ids = lax.broadcasted_iota(jnp.int32, (BLK, K), 1)
    idx = jnp.min(jnp.where(dist == minv, ids, K), axis=1)
    oh = (ids == idx[:, None]).astype(jnp.float32)
    q = lax.dot_general(oh, e, (((1,), (0,)), ((), ())),
                        preferred_element_type=jnp.float32)
    out_ref[...] = x + (q - x)
    diff = q - x
    s = jnp.sum(diff * diff)

    @pl.when(i == 0)
    def _init():
        loss_ref[...] = jnp.zeros_like(loss_ref)

    loss_ref[...] += jnp.reshape(s, (1, 1))


def kernel(latents, embedding_weight):
    lat = jnp.transpose(latents, (0, 2, 3, 1))
    shp = lat.shape
    flat = lat.reshape(-1, D)
    n = flat.shape[0]
    out_flat, loss = pl.pallas_call(
        _vq_block,
        grid=(n // BLK,),
        in_specs=[pl.BlockSpec((BLK, D), lambda i: (i, 0)),
                  pl.BlockSpec((K, D), lambda i: (0, 0))],
        out_specs=[pl.BlockSpec((BLK, D), lambda i: (i, 0)),
                   pl.BlockSpec((1, 1), lambda i: (0, 0))],
        out_shape=[jax.ShapeDtypeStruct((n, D), jnp.float32),
                   jax.ShapeDtypeStruct((1, 1), jnp.float32)],
    )(flat, embedding_weight)
    l = loss[0, 0] / (n * D)
    out = jnp.transpose(out_flat.reshape(shp), (0, 3, 1, 2))
    return (out, l * BETA, l)


# speedup vs baseline: 38.4501x; 1.0032x over previous
import jax
import jax.numpy as jnp
from jax import lax
from jax.experimental import pallas as pl

K = 1024
D = 64
BETA = 0.25
BLK = 512


def _vq_block(flat_ref, emb_ref, out_ref, loss_ref):
    i = pl.program_id(0)
    x = flat_ref[...]
    e = emb_ref[...]
    m = lax.dot_general(x, e, (((1,), (1,)), ((), ())),
                        preferred_element_type=jnp.float32)
    sx = jnp.sum(x ** 2, axis=1, keepdims=True)
    se = jnp.sum(e ** 2, axis=1)
    dist = sx + se[None, :] - 2.0 * m
    minv = jnp.min(dist, axis=1, keepdims=True)
    ids = lax.broadcasted_iota(jnp.int32, (BLK, K), 1)
    idx = jnp.min(jnp.where(dist == minv, ids, K), axis=1)
    oh = (ids == idx[:, None]).astype(jnp.bfloat16)
    q = lax.dot_general(oh, e.astype(jnp.bfloat16),
                        (((1,), (0,)), ((), ())),
                        preferred_element_type=jnp.float32)
    out_ref[...] = x + (q - x)
    diff = q - x
    s = jnp.sum(diff * diff)

    @pl.when(i == 0)
    def _init():
        loss_ref[...] = jnp.zeros_like(loss_ref)

    loss_ref[...] += jnp.reshape(s, (1, 1))


def kernel(latents, embedding_weight):
    lat = jnp.transpose(latents, (0, 2, 3, 1))
    shp = lat.shape
    flat = lat.reshape(-1, D)
    n = flat.shape[0]
    out_flat, loss = pl.pallas_call(
        _vq_block,
        grid=(n // BLK,),
        in_specs=[pl.BlockSpec((BLK, D), lambda i: (i, 0)),
                  pl.BlockSpec((K, D), lambda i: (0, 0))],
        out_specs=[pl.BlockSpec((BLK, D), lambda i: (i, 0)),
                   pl.BlockSpec((1, 1), lambda i: (0, 0))],
        out_shape=[jax.ShapeDtypeStruct((n, D), jnp.float32),
                   jax.ShapeDtypeStruct((1, 1), jnp.float32)],
    )(flat, embedding_weight)
    l = loss[0, 0] / (n * D)
    out = jnp.transpose(out_flat.reshape(shp), (0, 3, 1, 2))
    return (out, l * BETA, l)


# speedup vs baseline: 40.0875x; 1.0426x over previous
import jax
import jax.numpy as jnp
from jax import lax
from jax.experimental import pallas as pl
from jax.experimental.pallas import tpu as pltpu

K = 1024
D = 64
BETA = 0.25
BLK = 512


def _vq_block(flat_ref, emb_ref, out_ref, loss_ref, se_ref):
    i = pl.program_id(0)
    x = flat_ref[...]
    e = emb_ref[...]

    @pl.when(i == 0)
    def _init():
        se_ref[...] = jnp.sum(e ** 2, axis=1)[None, :]
        loss_ref[...] = jnp.zeros_like(loss_ref)

    m = lax.dot_general(x, e, (((1,), (1,)), ((), ())),
                        preferred_element_type=jnp.float32)
    sx = jnp.sum(x ** 2, axis=1, keepdims=True)
    dist = sx + se_ref[...] - 2.0 * m
    minv = jnp.min(dist, axis=1, keepdims=True)
    ids = lax.broadcasted_iota(jnp.int32, (BLK, K), 1)
    idx = jnp.min(jnp.where(dist == minv, ids, K), axis=1)
    oh = (ids == idx[:, None]).astype(jnp.float32)
    out_ref[...] = lax.dot_general(oh, e, (((1,), (0,)), ((), ())),
                                   preferred_element_type=jnp.float32)
    loss_ref[...] += jnp.reshape(jnp.sum(minv), (1, 1))


def kernel(latents, embedding_weight):
    lat = jnp.transpose(latents, (0, 2, 3, 1))
    shp = lat.shape
    flat = lat.reshape(-1, D)
    n = flat.shape[0]
    out_flat, loss = pl.pallas_call(
        _vq_block,
        grid=(n // BLK,),
        in_specs=[pl.BlockSpec((BLK, D), lambda i: (i, 0)),
                  pl.BlockSpec((K, D), lambda i: (0, 0))],
        out_specs=[pl.BlockSpec((BLK, D), lambda i: (i, 0)),
                   pl.BlockSpec((1, 1), lambda i: (0, 0))],
        out_shape=[jax.ShapeDtypeStruct((n, D), jnp.float32),
                   jax.ShapeDtypeStruct((1, 1), jnp.float32)],
        scratch_shapes=[pltpu.VMEM((1, K), jnp.float32)],
    )(flat, embedding_weight)
    l = loss[0, 0] / (n * D)
    out = jnp.transpose(out_flat.reshape(shp), (0, 3, 1, 2))
    return (out, l * BETA, l)
